# baseline (speedup 1.0000x reference)
import jax
import jax.numpy as jnp
from jax import lax
from jax.experimental import pallas as pl
from jax.experimental.pallas import tpu as pltpu

T = 2048
D = 1024
E = 8
B = 128
NB = T // B
NBUF = 3
CK = 8


def _gates_kernel(x_ref, w_ref, gates_ref, ent_ref):
    x = x_ref[...]
    w = w_ref[...]
    logits = lax.dot_general(x, w, (((1,), (1,)), ((), ())),
                             preferred_element_type=jnp.float32)
    m = jnp.max(logits, axis=1, keepdims=True)
    ex = jnp.exp(logits - m)
    g = ex / jnp.sum(ex, axis=1, keepdims=True)
    gates_ref[...] = g
    ent = -jnp.sum(g * jnp.log(g + 1e-9), axis=1)
    ent_ref[0, 0] = jnp.sum(ent) / jnp.float32(T)


def _diag_kernel(gates_ref, comb_ref, mask_ref, cbuf, csem):
    i = pl.program_id(0)
    b = lax.rem(i, NBUF)

    def sub_copies(buf_idx, step):
        bk = B // CK
        return [
            pltpu.make_async_copy(
                cbuf.at[buf_idx, pl.ds(k * bk, bk)],
                comb_ref.at[pl.ds(step * B + k * bk, bk)],
                csem.at[buf_idx],
            )
            for k in range(CK)
        ]

    @pl.when(i >= NBUF)
    def _reclaim():
        for c in sub_copies(b, i - NBUF):
            c.wait()
        cbuf[b, :, :, pl.ds((i - NBUF) * B, B)] = jnp.zeros(
            (B, E, B), jnp.float32)

    @pl.when(i < NBUF)
    def _init():
        cbuf[b] = jnp.zeros((B, E, T), jnp.float32)

    g = gates_ref[pl.ds(i * B, B), :]
    row = lax.broadcasted_iota(jnp.int32, (B, E, B), 0)
    col = lax.broadcasted_iota(jnp.int32, (B, E, B), 2)
    d = row == col
    gb = g[:, :, None]
    cbuf[b, :, :, pl.ds(i * B, B)] = jnp.where(d, gb, 0.0)
    for c in sub_copies(b, i):
        c.start()

    mask_ref[...] = jnp.zeros((B, E * T), jnp.bool_)
    row2 = lax.broadcasted_iota(jnp.int32, (B, B), 0)
    col2 = lax.broadcasted_iota(jnp.int32, (B, B), 1)
    d2 = row2 == col2
    for e in range(E):
        ge = g[:, e][:, None]
        mask_ref[:, pl.ds(e * T + i * B, B)] = jnp.logical_and(d2, ge != 0.0)

    @pl.when(i == NB - 1)
    def _drain():
        for s in range(NB - NBUF, NB):
            for c in sub_copies(s % NBUF, s):
                c.wait()


def kernel(input, W):
    gates, ent = pl.pallas_call(
        _gates_kernel,
        out_shape=(
            jax.ShapeDtypeStruct((T, E), jnp.float32),
            jax.ShapeDtypeStruct((1, 1), jnp.float32),
        ),
        out_specs=(
            pl.BlockSpec(memory_space=pltpu.VMEM),
            pl.BlockSpec(memory_space=pltpu.SMEM),
        ),
    )(input, W)

    comb, mask2 = pl.pallas_call(
        _diag_kernel,
        grid=(NB,),
        in_specs=(pl.BlockSpec(memory_space=pltpu.VMEM),),
        out_specs=(
            pl.BlockSpec(memory_space=pl.ANY),
            pl.BlockSpec((B, E * T), lambda i: (i, 0)),
        ),
        out_shape=(
            jax.ShapeDtypeStruct((T, E, T), jnp.float32),
            jax.ShapeDtypeStruct((T, E * T), jnp.bool_),
        ),
        scratch_shapes=[
            pltpu.VMEM((NBUF, B, E, T), jnp.float32),
            pltpu.SemaphoreType.DMA((NBUF,)),
        ],
    )(gates)

    mask = mask2.reshape(T, E, T)
    l_aux = jnp.zeros((1,), dtype=jnp.float32)
    return (l_aux, comb, mask, ent[0, 0])
